# Initial kernel scaffold; baseline (speedup 1.0000x reference)
import jax
import jax.numpy as jnp
from jax.experimental import pallas as pl


def kernel(x, W):
    raise NotImplementedError("write your pallas kernel here")


# speedup vs baseline: 4.3458x; 4.3458x over previous
import functools

import jax
import jax.numpy as jnp
from jax.experimental import pallas as pl
from jax.experimental.pallas import tpu as pltpu

_E = 64
_TB = 512


def _router_body(x_ref, wt_ref, imp_ref, ind_ref, loss_ref, me_ref, ce_ref,
                 *, n_tokens):
    i = pl.program_id(0)

    @pl.when(i == 0)
    def _init():
        me_ref[...] = jnp.zeros_like(me_ref)
        ce_ref[...] = jnp.zeros_like(ce_ref)

    logits = jnp.dot(x_ref[...], wt_ref[...],
                     preferred_element_type=jnp.float32)
    m = jnp.max(logits, axis=-1, keepdims=True)
    ex = jnp.exp(logits - m)
    s = jnp.sum(ex, axis=-1, keepdims=True)
    p = ex / s

    a1 = jnp.argmax(p, axis=-1)
    eiota = jax.lax.broadcasted_iota(jnp.int32, p.shape, 1)
    mask1 = eiota == a1[:, None]
    a2 = jnp.argmax(jnp.where(mask1, -1.0, p), axis=-1)
    mask2 = eiota == a2[:, None]
    sel = mask1 | mask2

    imp_ref[...] = jnp.where(sel, p, 0.0).T
    ind_ref[...] = sel.astype(jnp.float32).T

    me_ref[...] += jnp.sum(p, axis=0, keepdims=True)
    ce_ref[...] += jnp.sum(mask1.astype(jnp.float32), axis=0, keepdims=True)

    @pl.when(i == pl.num_programs(0) - 1)
    def _fini():
        scale = _E / float(n_tokens * n_tokens)
        loss_ref[0, 0] = jnp.sum(me_ref[...] * ce_ref[...]) * scale


def kernel(x, W):
    B, S, D = x.shape
    n = B * S
    xf = x.reshape(n, D)
    wt = W.T
    grid = n // _TB

    imp, ind, loss = pl.pallas_call(
        functools.partial(_router_body, n_tokens=n),
        grid=(grid,),
        in_specs=[
            pl.BlockSpec((_TB, D), lambda i: (i, 0)),
            pl.BlockSpec((D, _E), lambda i: (0, 0)),
        ],
        out_specs=[
            pl.BlockSpec((_E, _TB), lambda i: (0, i)),
            pl.BlockSpec((_E, _TB), lambda i: (0, i)),
            pl.BlockSpec(memory_space=pltpu.SMEM),
        ],
        out_shape=[
            jax.ShapeDtypeStruct((_E, n), jnp.float32),
            jax.ShapeDtypeStruct((_E, n), jnp.float32),
            jax.ShapeDtypeStruct((1, 1), jnp.float32),
        ],
        scratch_shapes=[
            pltpu.VMEM((1, _E), jnp.float32),
            pltpu.VMEM((1, _E), jnp.float32),
        ],
    )(xf, wt)

    imp = imp.reshape(_E, B, S, 1)
    ind = ind.reshape(_E, B, S, 1)
    return imp, ind, loss[0, 0]
